# initial kernel scaffold (unmeasured)
import functools

import jax
import jax.numpy as jnp
from jax import lax
from jax.experimental import pallas as pl
from jax.experimental.pallas import tpu as pltpu

N_DEV = 4
N_CHUNKS = 2


def kernel(x, w_mat, scale_x, scale_w):
    m_total, k_shard = x.shape
    k_total, n = w_mat.shape
    m_per = m_total // N_DEV
    n_chunk = n // N_CHUNKS

    def body(x_ref, w_ref, sx_ref, sw_ref, out_ref, comm_ref, send_sems, recv_sems):
        me = lax.axis_index("i")

        barrier_sem = pltpu.get_barrier_semaphore()
        for d in range(1, N_DEV):
            peer = lax.rem(me + d, N_DEV)
            pl.semaphore_signal(
                barrier_sem, inc=1,
                device_id=(peer,), device_id_type=pl.DeviceIdType.MESH,
            )
        pl.semaphore_wait(barrier_sem, N_DEV - 1)

        sends = []
        for d in (1, 3, 2):
            t = lax.rem(me + d, N_DEV)
            rdma = pltpu.make_async_remote_copy(
                src_ref=x_ref.at[pl.ds(t * m_per, m_per), :],
                dst_ref=comm_ref.at[me],
                send_sem=send_sems.at[d],
                recv_sem=recv_sems.at[me],
                device_id=(t,),
                device_id_type=pl.DeviceIdType.MESH,
            )
            rdma.start()
            sends.append(rdma)

        def accum(x_blk, k_idx, first):
            w_blk = w_ref[pl.ds(k_idx * k_shard, k_shard), :]
            for c in range(N_CHUNKS):
                ns = pl.ds(c * n_chunk, n_chunk)
                part = jnp.dot(
                    x_blk, w_blk[:, ns], preferred_element_type=jnp.int32
                ).astype(jnp.float32)
                if first:
                    out_ref[:, ns] = part
                else:
                    out_ref[:, ns] += part

        accum(x_ref[pl.ds(me * m_per, m_per), :], me, first=True)

        for off in (N_DEV - 1, 1, 2):
            j = lax.rem(me + off, N_DEV)
            recv = pltpu.make_async_remote_copy(
                src_ref=comm_ref.at[j],
                dst_ref=comm_ref.at[j],
                send_sem=send_sems.at[0],
                recv_sem=recv_sems.at[j],
                device_id=(j,),
                device_id_type=pl.DeviceIdType.MESH,
            )
            recv.wait_recv()
            accum(comm_ref[j], j, first=False)

        s = sx_ref[0] * sw_ref[0]
        out_ref[...] = jnp.maximum(out_ref[...] * s, 0.0)

        for rdma in sends:
            rdma.wait_send()

    return pl.pallas_call(
        body,
        out_shape=jax.ShapeDtypeStruct((m_per, n), jnp.float32),
        in_specs=[
            pl.BlockSpec(memory_space=pltpu.VMEM),
            pl.BlockSpec(memory_space=pltpu.VMEM),
            pl.BlockSpec(memory_space=pltpu.SMEM),
            pl.BlockSpec(memory_space=pltpu.SMEM),
        ],
        out_specs=pl.BlockSpec(memory_space=pltpu.VMEM),
        scratch_shapes=[
            pltpu.VMEM((N_DEV, m_per, k_shard), jnp.int8),
            pltpu.SemaphoreType.DMA((N_DEV,)),
            pltpu.SemaphoreType.DMA((N_DEV,)),
        ],
        compiler_params=pltpu.CompilerParams(collective_id=0),
    )(x, w_mat, scale_x, scale_w)


# baseline (device time: 121202 ns/iter reference)
import jax
import jax.numpy as jnp
from jax import lax
from jax.experimental import pallas as pl
from jax.experimental.pallas import tpu as pltpu

N_DEV = 4
N_CHUNK = 1024


def kernel(x, w_mat, scale_x, scale_w):
    m_total, k_shard = x.shape
    k_total, n = w_mat.shape
    m_per = m_total // N_DEV
    n_chunks = n // N_CHUNK

    def body(x_ref, w_hbm, sx_ref, sw_ref, out_ref,
             comm_ref, wbuf, send_sems, recv_sems, wsems):
        me = lax.axis_index("i")

        barrier_sem = pltpu.get_barrier_semaphore()
        for d in range(1, N_DEV):
            peer = lax.rem(me + d, N_DEV)
            pl.semaphore_signal(
                barrier_sem, inc=1,
                device_id=(peer,), device_id_type=pl.DeviceIdType.MESH,
            )
        pl.semaphore_wait(barrier_sem, N_DEV - 1)

        sends = []
        for d in (1, 3, 2):
            t = lax.rem(me + d, N_DEV)
            rdma = pltpu.make_async_remote_copy(
                src_ref=x_ref.at[pl.ds(t * m_per, m_per), :],
                dst_ref=comm_ref.at[me],
                send_sem=send_sems.at[d],
                recv_sem=recv_sems.at[me],
                device_id=(t,),
                device_id_type=pl.DeviceIdType.MESH,
            )
            rdma.start()
            sends.append(rdma)

        pairs = []
        for off in (0, N_DEV - 1, 1, 2):
            j = lax.rem(me + off, N_DEV)
            for c in range(n_chunks):
                pairs.append((off, j, c))

        def w_dma(q):
            _, j, c = pairs[q]
            return pltpu.make_async_copy(
                w_hbm.at[pl.ds(j * k_shard, k_shard),
                         pl.ds(c * N_CHUNK, N_CHUNK)],
                wbuf.at[q % 2],
                wsems.at[q % 2],
            )

        w_dma(0).start()
        for q, (off, j, c) in enumerate(pairs):
            ns = slice(c * N_CHUNK, (c + 1) * N_CHUNK)
            if c == 0 and off != 0:
                recv = pltpu.make_async_remote_copy(
                    src_ref=comm_ref.at[j],
                    dst_ref=comm_ref.at[j],
                    send_sem=send_sems.at[0],
                    recv_sem=recv_sems.at[j],
                    device_id=(j,),
                    device_id_type=pl.DeviceIdType.MESH,
                )
                recv.wait_recv()
            w_dma(q).wait()
            if q + 1 < len(pairs):
                w_dma(q + 1).start()
            if off == 0:
                x_blk = x_ref[pl.ds(me * m_per, m_per), :]
            else:
                x_blk = comm_ref[j]
            part = jnp.dot(
                x_blk, wbuf[q % 2], preferred_element_type=jnp.int32
            ).astype(jnp.float32)
            if off == 0:
                out_ref[:, ns] = part
            elif off == 2:
                s = sx_ref[0] * sw_ref[0]
                out_ref[:, ns] = jnp.maximum((out_ref[:, ns] + part) * s, 0.0)
            else:
                out_ref[:, ns] += part

        for rdma in sends:
            rdma.wait_send()

    return pl.pallas_call(
        body,
        out_shape=jax.ShapeDtypeStruct((m_per, n), jnp.float32),
        in_specs=[
            pl.BlockSpec(memory_space=pltpu.VMEM),
            pl.BlockSpec(memory_space=pltpu.MemorySpace.HBM),
            pl.BlockSpec(memory_space=pltpu.SMEM),
            pl.BlockSpec(memory_space=pltpu.SMEM),
        ],
        out_specs=pl.BlockSpec(memory_space=pltpu.VMEM),
        scratch_shapes=[
            pltpu.VMEM((N_DEV, m_per, k_shard), jnp.int8),
            pltpu.VMEM((2, k_shard, N_CHUNK), jnp.int8),
            pltpu.SemaphoreType.DMA((N_DEV,)),
            pltpu.SemaphoreType.DMA((N_DEV,)),
            pltpu.SemaphoreType.DMA((2,)),
        ],
        compiler_params=pltpu.CompilerParams(
            collective_id=0,
            vmem_limit_bytes=50 * 1024 * 1024,
        ),
    )(x, w_mat, scale_x, scale_w)


# device time: 118749 ns/iter; 1.0207x vs baseline; 1.0207x over previous
import jax
import jax.numpy as jnp
from jax import lax
from jax.experimental import pallas as pl
from jax.experimental.pallas import tpu as pltpu

N_DEV = 4
K_STRIP = 512
N_CHUNK = 1024


def kernel(x, w_mat, scale_x, scale_w):
    m_total, k_shard = x.shape
    k_total, n = w_mat.shape
    m_per = m_total // N_DEV
    n_chunks = n // N_CHUNK

    def body(x_ref, w_hbm, sx_ref, sw_ref, out_ref,
             comm_ref, wbuf, send_sems, recv_sems, wsems):
        me = lax.axis_index("i")

        barrier_sem = pltpu.get_barrier_semaphore()
        for d in range(1, N_DEV):
            peer = lax.rem(me + d, N_DEV)
            pl.semaphore_signal(
                barrier_sem, inc=1,
                device_id=(peer,), device_id_type=pl.DeviceIdType.MESH,
            )
        pl.semaphore_wait(barrier_sem, N_DEV - 1)

        sends = []
        for d in (1, 3, 2):
            t = lax.rem(me + d, N_DEV)
            rdma = pltpu.make_async_remote_copy(
                src_ref=x_ref.at[pl.ds(t * m_per, m_per), :],
                dst_ref=comm_ref.at[me],
                send_sem=send_sems.at[d],
                recv_sem=recv_sems.at[me],
                device_id=(t,),
                device_id_type=pl.DeviceIdType.MESH,
            )
            rdma.start()
            sends.append(rdma)

        halves = k_shard // K_STRIP
        strips = []
        for off in (0, N_DEV - 1, 1, 2):
            j = lax.rem(me + off, N_DEV)
            for h in range(halves):
                strips.append((off, j, h))

        def w_dma(q):
            _, j, h = strips[q]
            return pltpu.make_async_copy(
                w_hbm.at[pl.ds(j * k_shard + h * K_STRIP, K_STRIP), :],
                wbuf.at[q % 2],
                wsems.at[q % 2],
            )

        n_strips = len(strips)
        last_q = n_strips - 1
        w_dma(0).start()
        for q, (off, j, h) in enumerate(strips):
            ks = slice(h * K_STRIP, (h + 1) * K_STRIP)
            if h == 0 and off != 0:
                recv = pltpu.make_async_remote_copy(
                    src_ref=comm_ref.at[j],
                    dst_ref=comm_ref.at[j],
                    send_sem=send_sems.at[0],
                    recv_sem=recv_sems.at[j],
                    device_id=(j,),
                    device_id_type=pl.DeviceIdType.MESH,
                )
                recv.wait_recv()
            w_dma(q).wait()
            if q + 1 < n_strips:
                w_dma(q + 1).start()
            if off == 0:
                x_blk = x_ref[pl.ds(me * m_per, m_per), ks]
            else:
                x_blk = comm_ref[j, :, ks]
            for c in range(n_chunks):
                ns = slice(c * N_CHUNK, (c + 1) * N_CHUNK)
                part = jnp.dot(
                    x_blk, wbuf[q % 2][:, ns],
                    preferred_element_type=jnp.int32,
                ).astype(jnp.float32)
                if q == 0:
                    out_ref[:, ns] = part
                elif q == last_q:
                    s = sx_ref[0] * sw_ref[0]
                    out_ref[:, ns] = jnp.maximum(
                        (out_ref[:, ns] + part) * s, 0.0)
                else:
                    out_ref[:, ns] += part

        for rdma in sends:
            rdma.wait_send()

    return pl.pallas_call(
        body,
        out_shape=jax.ShapeDtypeStruct((m_per, n), jnp.float32),
        in_specs=[
            pl.BlockSpec(memory_space=pltpu.VMEM),
            pl.BlockSpec(memory_space=pltpu.MemorySpace.HBM),
            pl.BlockSpec(memory_space=pltpu.SMEM),
            pl.BlockSpec(memory_space=pltpu.SMEM),
        ],
        out_specs=pl.BlockSpec(memory_space=pltpu.VMEM),
        scratch_shapes=[
            pltpu.VMEM((N_DEV, m_per, k_shard), jnp.int8),
            pltpu.VMEM((2, K_STRIP, n), jnp.int8),
            pltpu.SemaphoreType.DMA((N_DEV,)),
            pltpu.SemaphoreType.DMA((N_DEV,)),
            pltpu.SemaphoreType.DMA((2,)),
        ],
        compiler_params=pltpu.CompilerParams(
            collective_id=0,
            vmem_limit_bytes=56 * 1024 * 1024,
        ),
    )(x, w_mat, scale_x, scale_w)
